# baseline (speedup 1.0000x reference)
import functools

import jax
import jax.numpy as jnp
from jax import lax
from jax.experimental import pallas as pl
from jax.experimental.pallas import tpu as pltpu
from jax.experimental.pallas import tpu_sc as plsc

BATCH = 16384
EMBED_DIM = 128
LANES = 16
CHUNK = 128


def _build_sc_call():
    info = plsc.get_sparse_core_info()
    nc, ns = info.num_cores, info.num_subcores
    nw = nc * ns
    bpw = BATCH // nw
    n_chunks = bpw // CHUNK
    groups_per_chunk = CHUNK // LANES
    n_groups = bpw // LANES

    mesh = plsc.VectorSubcoreMesh(core_axis_name="c", subcore_axis_name="s")

    @functools.partial(
        pl.kernel,
        mesh=mesh,
        compiler_params=pltpu.CompilerParams(
            needs_layout_passes=False,
            disable_bounds_checks=True,
        ),
        out_type=jax.ShapeDtypeStruct((BATCH,), jnp.float32),
        scratch_types=[
            pltpu.VMEM((bpw,), jnp.int32),
            pltpu.VMEM((bpw,), jnp.int32),
            pltpu.VMEM((bpw,), jnp.float32),
            pltpu.VMEM((bpw,), jnp.float32),
            pltpu.VMEM((1,), jnp.float32),
            pltpu.VMEM((CHUNK, EMBED_DIM), jnp.float32),
            pltpu.VMEM((CHUNK, EMBED_DIM), jnp.float32),
            pltpu.VMEM((CHUNK, EMBED_DIM), jnp.float32),
            pltpu.VMEM((CHUNK, EMBED_DIM), jnp.float32),
            pltpu.VMEM((CHUNK, EMBED_DIM), jnp.float32),
            pltpu.VMEM((CHUNK, EMBED_DIM), jnp.float32),
            pltpu.VMEM((CHUNK // LANES, LANES, LANES), jnp.float32),
            pltpu.VMEM((bpw,), jnp.float32),
            pltpu.SemaphoreType.DMA,
            pltpu.SemaphoreType.DMA,
            pltpu.SemaphoreType.DMA,
            pltpu.SemaphoreType.DMA,
        ],
    )
    def sc_call(uidx_hbm, iidx_hbm, utab_hbm, itab_hbm, ubias_hbm, ibias_hbm,
                gbias_hbm, out_hbm, uidx_v, iidx_v, ub_v, ib_v, gb1,
                urows0, urows1, urows2, irows0, irows1, irows2, accbuf, out_v,
                sem0, sem1, sem2, sem_b):
        wid = lax.axis_index("s") * nc + lax.axis_index("c")
        base = wid * bpw

        pltpu.sync_copy(uidx_hbm.at[pl.ds(base, bpw)], uidx_v)
        pltpu.sync_copy(iidx_hbm.at[pl.ds(base, bpw)], iidx_v)
        pltpu.sync_copy(gbias_hbm, gb1)
        cb_u = pltpu.async_copy(ubias_hbm.at[0].at[uidx_v], ub_v, sem_b)
        cb_i = pltpu.async_copy(ibias_hbm.at[0].at[iidx_v], ib_v, sem_b)

        ubufs = (urows0, urows1, urows2)
        ibufs = (irows0, irows1, irows2)
        sems = (sem0, sem1, sem2)
        nbuf = 3
        lane = lax.iota(jnp.int32, LANES)
        last_lane = lane == (LANES - 1)
        zero16 = jnp.zeros((LANES,), jnp.int32)

        def fire(c):
            p = c % nbuf
            cu = pltpu.async_copy(
                utab_hbm.at[uidx_v.at[pl.ds(c * CHUNK, CHUNK)]], ubufs[p],
                sems[p])
            ci = pltpu.async_copy(
                itab_hbm.at[iidx_v.at[pl.ds(c * CHUNK, CHUNK)]], ibufs[p],
                sems[p])
            return cu, ci

        inflight = [fire(c) for c in range(min(nbuf, n_chunks))]
        for c in range(n_chunks):
            cu, ci = inflight.pop(0)
            cu.wait()
            ci.wait()
            ur, ir = ubufs[c % nbuf], ibufs[c % nbuf]

            @plsc.parallel_loop(0, groups_per_chunk, 1)
            def group_body(g, ur=ur, ir=ir, c=c):
                out_base = pl.multiple_of(
                    (c * groups_per_chunk + g) * LANES, LANES)
                s0 = g * LANES

                @plsc.parallel_loop(0, LANES, 1, unroll=2)
                def sample_body(sl):
                    s = s0 + sl
                    acc = ur[s, pl.ds(0, LANES)] * ir[s, pl.ds(0, LANES)]
                    for j in range(1, EMBED_DIM // LANES):
                        acc = acc + (ur[s, pl.ds(j * LANES, LANES)]
                                     * ir[s, pl.ds(j * LANES, LANES)])
                    accbuf[g, sl, :] = acc

                gfull = jnp.full((LANES,), 0, jnp.int32) + g
                vecs = [plsc.load_gather(
                            accbuf,
                            [gfull, lane, jnp.full((LANES,), j, jnp.int32)])
                        for j in range(LANES)]
                while len(vecs) > 1:
                    vecs = [vecs[i] + vecs[i + 1]
                            for i in range(0, len(vecs), 2)]
                out_v[pl.ds(out_base, LANES)] = vecs[0]

            if c + nbuf < n_chunks:
                inflight.append(fire(c + nbuf))

        cb_u.wait()
        cb_i.wait()
        gb = plsc.load_gather(gb1, [zero16])
        for r in range(n_groups):
            out_v[pl.ds(r * LANES, LANES)] = (
                out_v[pl.ds(r * LANES, LANES)] + ub_v[pl.ds(r * LANES, LANES)]
                + ib_v[pl.ds(r * LANES, LANES)] + gb)
        pltpu.sync_copy(out_v, out_hbm.at[pl.ds(base, bpw)])

    return sc_call


def kernel(user_idx, item_idx, user_table, item_table, user_bias, item_bias,
           global_bias):
    uidx = user_idx.astype(jnp.int32)
    iidx = item_idx.astype(jnp.int32)
    return _build_sc_call()(uidx, iidx, user_table, item_table,
                            user_bias.T, item_bias.T, global_bias)


# speedup vs baseline: 1.0431x; 1.0431x over previous
import functools

import jax
import jax.numpy as jnp
from jax import lax
from jax.experimental import pallas as pl
from jax.experimental.pallas import tpu as pltpu
from jax.experimental.pallas import tpu_sc as plsc

BATCH = 16384
EMBED_DIM = 128
LANES = 16
CHUNK = 128


def _build_sc_call():
    info = plsc.get_sparse_core_info()
    nc, ns = info.num_cores, info.num_subcores
    nw = nc * ns
    bpw = BATCH // nw
    n_chunks = bpw // CHUNK
    groups_per_chunk = CHUNK // LANES
    n_groups = bpw // LANES

    mesh = plsc.VectorSubcoreMesh(core_axis_name="c", subcore_axis_name="s")

    @functools.partial(
        pl.kernel,
        mesh=mesh,
        compiler_params=pltpu.CompilerParams(
            needs_layout_passes=False,
            disable_bounds_checks=True,
        ),
        out_type=jax.ShapeDtypeStruct((BATCH,), jnp.float32),
        scratch_types=[
            pltpu.VMEM((bpw,), jnp.int32),
            pltpu.VMEM((bpw,), jnp.int32),
            pltpu.VMEM((bpw,), jnp.float32),
            pltpu.VMEM((bpw,), jnp.float32),
            pltpu.VMEM((1,), jnp.float32),
            pltpu.VMEM((CHUNK, EMBED_DIM), jnp.float32),
            pltpu.VMEM((CHUNK, EMBED_DIM), jnp.float32),
            pltpu.VMEM((CHUNK, EMBED_DIM), jnp.float32),
            pltpu.VMEM((CHUNK, EMBED_DIM), jnp.float32),
            pltpu.VMEM((CHUNK // LANES, LANES, LANES), jnp.float32),
            pltpu.VMEM((bpw,), jnp.float32),
            pltpu.SemaphoreType.DMA,
            pltpu.SemaphoreType.DMA,
            pltpu.SemaphoreType.DMA,
        ],
    )
    def sc_call(uidx_hbm, iidx_hbm, utab_hbm, itab_hbm, ubias_hbm, ibias_hbm,
                gbias_hbm, out_hbm, uidx_v, iidx_v, ub_v, ib_v, gb1,
                urows0, urows1, irows0, irows1, accbuf, out_v,
                sem0, sem1, sem_b):
        wid = lax.axis_index("s") * nc + lax.axis_index("c")
        base = wid * bpw

        pltpu.sync_copy(uidx_hbm.at[pl.ds(base, bpw)], uidx_v)
        pltpu.sync_copy(iidx_hbm.at[pl.ds(base, bpw)], iidx_v)
        pltpu.sync_copy(gbias_hbm, gb1)
        cb_u = pltpu.async_copy(ubias_hbm.at[0].at[uidx_v], ub_v, sem_b)
        cb_i = pltpu.async_copy(ibias_hbm.at[0].at[iidx_v], ib_v, sem_b)

        ubufs = (urows0, urows1)
        ibufs = (irows0, irows1)
        sems = (sem0, sem1)
        lane = lax.iota(jnp.int32, LANES)
        last_lane = lane == (LANES - 1)
        zero16 = jnp.zeros((LANES,), jnp.int32)

        def fire(c):
            p = c % 2
            cu = pltpu.async_copy(
                utab_hbm.at[uidx_v.at[pl.ds(c * CHUNK, CHUNK)]], ubufs[p],
                sems[p])
            ci = pltpu.async_copy(
                itab_hbm.at[iidx_v.at[pl.ds(c * CHUNK, CHUNK)]], ibufs[p],
                sems[p])
            return cu, ci

        inflight = fire(0)
        for c in range(n_chunks):
            nxt = fire(c + 1) if c + 1 < n_chunks else None
            inflight[0].wait()
            inflight[1].wait()
            ur, ir = ubufs[c % 2], ibufs[c % 2]

            @plsc.parallel_loop(0, groups_per_chunk, 1)
            def group_body(g, ur=ur, ir=ir, c=c):
                out_base = pl.multiple_of(
                    (c * groups_per_chunk + g) * LANES, LANES)
                s0 = g * LANES

                @plsc.parallel_loop(0, LANES, 1, unroll=2)
                def sample_body(sl):
                    s = s0 + sl
                    acc = ur[s, pl.ds(0, LANES)] * ir[s, pl.ds(0, LANES)]
                    for j in range(1, EMBED_DIM // LANES):
                        acc = acc + (ur[s, pl.ds(j * LANES, LANES)]
                                     * ir[s, pl.ds(j * LANES, LANES)])
                    accbuf[g, sl, :] = acc

                gfull = jnp.full((LANES,), 0, jnp.int32) + g
                vecs = [plsc.load_gather(
                            accbuf,
                            [gfull, lane, jnp.full((LANES,), j, jnp.int32)])
                        for j in range(LANES)]
                while len(vecs) > 1:
                    vecs = [vecs[i] + vecs[i + 1]
                            for i in range(0, len(vecs), 2)]
                out_v[pl.ds(out_base, LANES)] = vecs[0]
            inflight = nxt

        cb_u.wait()
        cb_i.wait()
        gb = plsc.load_gather(gb1, [zero16])
        for r in range(n_groups):
            out_v[pl.ds(r * LANES, LANES)] = (
                out_v[pl.ds(r * LANES, LANES)] + ub_v[pl.ds(r * LANES, LANES)]
                + ib_v[pl.ds(r * LANES, LANES)] + gb)
        pltpu.sync_copy(out_v, out_hbm.at[pl.ds(base, bpw)])

    return sc_call


def kernel(user_idx, item_idx, user_table, item_table, user_bias, item_bias,
           global_bias):
    uidx = user_idx.astype(jnp.int32)
    iidx = item_idx.astype(jnp.int32)
    return _build_sc_call()(uidx, iidx, user_table, item_table,
                            user_bias.T, item_bias.T, global_bias)
